# baseline (speedup 1.0000x reference)
import functools

import jax
import jax.numpy as jnp
from jax import lax
from jax.experimental import pallas as pl
from jax.experimental.pallas import tpu as pltpu
from jax.experimental.pallas import tpu_sc as plsc

NUM_EMB = 1000000
DIM = 64
SEQ = 16384
NUM_WORKERS = 32
B_PER_W = SEQ // NUM_WORKERS


def _body(table_hbm, idx_hbm, out_hbm, idx_v, rows_v, sem):
    wid = lax.axis_index("s") * 2 + lax.axis_index("c")
    base = wid * B_PER_W
    pltpu.sync_copy(idx_hbm.at[pl.ds(base, B_PER_W)], idx_v)
    pltpu.async_copy(table_hbm.at[idx_v], rows_v, sem).wait()
    pltpu.sync_copy(rows_v, out_hbm.at[pl.ds(base, B_PER_W)])


def kernel(token_ids, embedding_matrix):
    mesh = plsc.VectorSubcoreMesh(core_axis_name="c", subcore_axis_name="s")
    k = pl.kernel(
        _body,
        mesh=mesh,
        out_type=jax.ShapeDtypeStruct((SEQ, DIM), jnp.float32),
        scratch_types=[
            pltpu.VMEM((B_PER_W,), jnp.int32),
            pltpu.VMEM((B_PER_W, DIM), jnp.float32),
            pltpu.SemaphoreType.DMA,
        ],
        compiler_params=pltpu.CompilerParams(use_tc_tiling_on_sc=False),
    )
    return k(embedding_matrix, token_ids.astype(jnp.int32))


# speedup vs baseline: 1.6308x; 1.6308x over previous
import functools

import jax
import jax.numpy as jnp
from jax import lax
from jax.experimental import pallas as pl
from jax.experimental.pallas import tpu as pltpu
from jax.experimental.pallas import tpu_sc as plsc

NUM_EMB = 1000000
DIM = 64
SEQ = 16384
NUM_WORKERS = 32
B_PER_W = SEQ // NUM_WORKERS
FLIGHT = 16


def _body(table_hbm, idx_hbm, out_hbm, idx_v, rows_v, gsem, dsem):
    wid = lax.axis_index("s") * 2 + lax.axis_index("c")
    base = wid * B_PER_W
    pltpu.sync_copy(idx_hbm.at[pl.ds(base, B_PER_W)], idx_v)

    def group(g, _):
        vec = idx_v[pl.ds(g * FLIGHT, FLIGHT)]
        for i in range(FLIGHT):
            row = vec[i]
            pltpu.async_copy(
                table_hbm.at[pl.ds(row, 1), :],
                rows_v.at[pl.ds(g * FLIGHT + i, 1), :],
                dsem,
            )
        for i in range(FLIGHT):
            pltpu.make_async_copy(
                table_hbm.at[pl.ds(0, 1), :],
                rows_v.at[pl.ds(i, 1), :],
                dsem,
            ).wait()
        return ()

    lax.fori_loop(0, B_PER_W // FLIGHT, group, ())
    pltpu.sync_copy(rows_v, out_hbm.at[pl.ds(base, B_PER_W)])


def kernel(token_ids, embedding_matrix):
    mesh = plsc.VectorSubcoreMesh(core_axis_name="c", subcore_axis_name="s")
    k = pl.kernel(
        _body,
        mesh=mesh,
        out_type=jax.ShapeDtypeStruct((SEQ, DIM), jnp.float32),
        scratch_types=[
            pltpu.VMEM((B_PER_W,), jnp.int32),
            pltpu.VMEM((B_PER_W, DIM), jnp.float32),
            pltpu.SemaphoreType.DMA,
            pltpu.SemaphoreType.DMA,
        ],
    )
    return k(embedding_matrix, token_ids.astype(jnp.int32))


# speedup vs baseline: 1.7014x; 1.0433x over previous
import functools

import jax
import jax.numpy as jnp
from jax import lax
from jax.experimental import pallas as pl
from jax.experimental.pallas import tpu as pltpu
from jax.experimental.pallas import tpu_sc as plsc

NUM_EMB = 1000000
DIM = 64
SEQ = 16384
NUM_WORKERS = 32
B_PER_W = SEQ // NUM_WORKERS
FLIGHT = 32
NG = B_PER_W // FLIGHT


def _body(table_hbm, idx_hbm, out_hbm, idx_v, rows_v, dsem):
    wid = lax.axis_index("s") * 2 + lax.axis_index("c")
    base = wid * B_PER_W
    pltpu.sync_copy(idx_hbm.at[pl.ds(base, B_PER_W)], idx_v)

    def fire(g):
        gb = g * FLIGHT
        for v16 in range(FLIGHT // 16):
            vec = idx_v[pl.ds(gb + v16 * 16, 16)]
            for i in range(16):
                row = vec[i]
                pltpu.async_copy(
                    table_hbm.at[pl.ds(row, 1), :],
                    rows_v.at[pl.ds(gb + v16 * 16 + i, 1), :],
                    dsem,
                )

    def drain(g):
        pltpu.make_async_copy(
            table_hbm.at[pl.ds(0, FLIGHT), :],
            rows_v.at[pl.ds(g * FLIGHT, FLIGHT), :],
            dsem,
        ).wait()

    fire(0)

    def group(g, _):
        fire(g + 1)
        drain(g)
        return ()

    lax.fori_loop(0, NG - 1, group, ())
    drain(NG - 1)
    pltpu.sync_copy(rows_v, out_hbm.at[pl.ds(base, B_PER_W)])


def kernel(token_ids, embedding_matrix):
    mesh = plsc.VectorSubcoreMesh(core_axis_name="c", subcore_axis_name="s")
    k = pl.kernel(
        _body,
        mesh=mesh,
        out_type=jax.ShapeDtypeStruct((SEQ, DIM), jnp.float32),
        scratch_types=[
            pltpu.VMEM((B_PER_W,), jnp.int32),
            pltpu.VMEM((B_PER_W, DIM), jnp.float32),
            pltpu.SemaphoreType.DMA,
        ],
    )
    return k(embedding_matrix, token_ids.astype(jnp.int32))
